# baseline (speedup 1.0000x reference)
import functools

import jax
import jax.numpy as jnp
from jax import lax
from jax.experimental import pallas as pl
from jax.experimental.pallas import tpu as pltpu
from jax.experimental.pallas import tpu_sc as plsc

N = 10000
NP = 10240
D = 128
NSUB = 16
NCALLS = 4
NHALF = NP // NCALLS
QTR = NHALF // 2
ZROW = NP - 1
CHUNK = 128
NCHUNK = 160
EPT = CHUNK * NCHUNK
STRIPE = QTR // NSUB
WSUB = 5
WROWS = STRIPE // WSUB
HSTRIPE = NP // NSUB
NBUF = 2
L = 16

_MESH = plsc.VectorSubcoreMesh(core_axis_name="c", subcore_axis_name="s")


def _deg_body(dst_hbm, zeros_hbm, out_hbm, dst_v, ones_v, hist_sh, sem):
    c = lax.axis_index("c")
    s = lax.axis_index("s")
    pltpu.sync_copy(zeros_hbm, hist_sh.at[pl.ds(s * HSTRIPE, HSTRIPE)])
    pltpu.async_copy(dst_hbm.at[s], dst_v, sem).wait()
    ones = jnp.full((L,), 1.0, dtype=jnp.float32)
    for i in range(CHUNK // L):
        ones_v[pl.ds(i * L, L)] = ones
    plsc.subcore_barrier()

    half = NCHUNK // 2

    def body(j, carry):
        pltpu.sync_copy(ones_v, hist_sh.at[dst_v.at[c * half + j]], add=True)
        return carry

    lax.fori_loop(0, half, body, 0)
    plsc.subcore_barrier()
    pltpu.sync_copy(hist_sh.at[pl.ds(s * HSTRIPE, HSTRIPE)],
                    out_hbm.at[c, pl.ds(s * HSTRIPE, HSTRIPE)])


@functools.partial(
    pl.kernel,
    mesh=_MESH,
    out_type=jax.ShapeDtypeStruct((2, NP), jnp.float32),
    scratch_types=[
        pltpu.VMEM((NCHUNK, CHUNK), jnp.int32),
        pltpu.VMEM((CHUNK,), jnp.float32),
        pltpu.VMEM_SHARED((NP,), jnp.float32),
        pltpu.SemaphoreType.DMA,
    ],
)
def _deg_kernel(dst_hbm, zeros_hbm, out_hbm, dst_v, ones_v, hist_sh, sem):
    _deg_body(dst_hbm, zeros_hbm, out_hbm, dst_v, ones_v, hist_sh, sem)


def _agg_body(hb, y_hbm, sf_hbm, df_hbm, out_hbm, sf_v, df_v, src_v, dst_v,
              bufs, fstage, pki_v, widx_v, agg_sh, sems, isem):
    c = lax.axis_index("c")
    s = lax.axis_index("s")
    ii = lax.iota(jnp.int32, L)
    zvec = jnp.zeros((L,), jnp.float32)

    def zrow_loop(r, carry):
        for k in range(D // L):
            fstage[r, pl.ds(k * L, L)] = zvec
        return carry

    lax.fori_loop(0, WROWS, zrow_loop, 0)
    for w in range(WSUB):
        pltpu.sync_copy(fstage,
                        agg_sh.at[pl.ds(s * STRIPE + w * WROWS, WROWS)])

    for t in range(2):
        for k in range(NCHUNK // (2 * L)):
            pki_v[t, pl.ds(k * L, L)] = ii + (s * NCHUNK + t * (NCHUNK // 2)
                                              + k * L)
    for t in range(2):
        pltpu.async_copy(sf_hbm.at[pki_v.at[t]],
                         sf_v.at[pl.ds(t * (NCHUNK // 2), NCHUNK // 2)],
                         isem).wait()
        pltpu.async_copy(df_hbm.at[pki_v.at[t]],
                         df_v.at[pl.ds(t * (NCHUNK // 2), NCHUNK // 2)],
                         isem).wait()

    lo = jnp.full((L,), hb * NHALF + c * QTR, dtype=jnp.int32)
    hi = jnp.full((L,), hb * NHALF + (c + 1) * QTR, dtype=jnp.int32)
    zrow = jnp.full((L,), ZROW, dtype=jnp.int32)
    zero = jnp.full((L,), 0, dtype=jnp.int32)

    def widen(j, carry):
        for k in range(CHUNK // L):
            col = k * L
            sv = sf_v[j, pl.ds(col, L)].astype(jnp.int32)
            v = df_v[j, pl.ds(col, L)].astype(jnp.int32)
            inr = (v >= lo) & (v < hi)
            src_v[j, pl.ds(col, L)] = jnp.where(inr, sv, zrow)
            dst_v[j, pl.ds(col, L)] = jnp.where(inr, v - lo, zero)
        return carry

    lax.fori_loop(0, NCHUNK, widen, 0)
    plsc.subcore_barrier()

    def body(g, carry):
        base = g * NBUF
        handles = [
            pltpu.async_copy(y_hbm.at[src_v.at[base + b]], bufs[b], sems[b])
            for b in range(NBUF)
        ]
        for b in range(NBUF):
            handles[b].wait()
            pltpu.sync_copy(bufs[b], agg_sh.at[dst_v.at[base + b]], add=True)
        return carry

    lax.fori_loop(0, NCHUNK // NBUF, body, 0)
    plsc.subcore_barrier()

    grow = c * QTR + s * STRIPE
    for w in range(WSUB):
        for k in range(WROWS // L):
            widx_v[w, pl.ds(k * L, L)] = ii + (grow + w * WROWS + k * L)
    for w in range(WSUB):
        pltpu.sync_copy(agg_sh.at[pl.ds(s * STRIPE + w * WROWS, WROWS)],
                        fstage)
        pltpu.sync_copy(fstage, out_hbm.at[widx_v.at[w]])


def _make_agg(hb):
    @functools.partial(
        pl.kernel,
        mesh=_MESH,
        out_type=jax.ShapeDtypeStruct((NHALF, D), jnp.float32),
        scratch_types=[
            pltpu.VMEM((NCHUNK, CHUNK), jnp.float32),
            pltpu.VMEM((NCHUNK, CHUNK), jnp.float32),
            pltpu.VMEM((NCHUNK, CHUNK), jnp.int32),
            pltpu.VMEM((NCHUNK, CHUNK), jnp.int32),
            pltpu.VMEM((CHUNK, D), jnp.float32),
            pltpu.VMEM((CHUNK, D), jnp.float32),
            pltpu.VMEM((WROWS, D), jnp.float32),
            pltpu.VMEM((2, NCHUNK // 2), jnp.int32),
            pltpu.VMEM((WSUB, WROWS), jnp.int32),
            pltpu.VMEM_SHARED((QTR, D), jnp.float32),
            pltpu.SemaphoreType.DMA,
            pltpu.SemaphoreType.DMA,
            pltpu.SemaphoreType.DMA,
        ],
    )
    def agg(y_hbm, sf_hbm, df_hbm, out_hbm, sf_v, df_v, src_v, dst_v,
            b0, b1, fstage, pki_v, widx_v, agg_sh, s0, s1, isem):
        _agg_body(hb, y_hbm, sf_hbm, df_hbm, out_hbm, sf_v, df_v,
                  src_v, dst_v, [b0, b1], fstage, pki_v, widx_v, agg_sh,
                  [s0, s1], isem)

    return agg


_aggs = [_make_agg(h) for h in range(NCALLS)]


_RB = 512
_GRID = NP // _RB


def _mm1_body(x_ref, w_ref, h_ref, y_ref):
    deg = h_ref[0, :] + h_ref[1, :] + 1.0
    dis = lax.rsqrt(deg)
    y_ref[...] = dis[:, None] * jnp.dot(
        x_ref[...], w_ref[...], preferred_element_type=jnp.float32)


def _mid_body(g_ref, y_ref, h_ref, b_ref, w2_ref, x1_ref, y2_ref):
    deg = h_ref[0, :] + h_ref[1, :] + 1.0
    dis = lax.rsqrt(deg)
    agg = g_ref[...] + y_ref[...]
    x1 = jnp.maximum(dis[:, None] * agg + b_ref[...][None, :], 0.0)
    x1_ref[...] = x1
    y2_ref[...] = dis[:, None] * jnp.dot(
        x1, w2_ref[...], preferred_element_type=jnp.float32)

    @pl.when(pl.program_id(0) == _GRID - 1)
    def _():
        y2_ref[_RB - 1, :] = jnp.zeros((D,), jnp.float32)


def _fin_body(g_ref, y_ref, h_ref, b_ref, x1_ref, wa_ref, wb_ref, bl_ref,
              o_ref):
    deg = h_ref[0, :] + h_ref[1, :] + 1.0
    dis = lax.rsqrt(deg)
    agg = g_ref[...] + y_ref[...]
    x2 = jnp.maximum(dis[:, None] * agg + b_ref[...][None, :], 0.0)
    o_ref[...] = (
        jnp.dot(x1_ref[...], wa_ref[...], preferred_element_type=jnp.float32)
        + jnp.dot(x2, wb_ref[...], preferred_element_type=jnp.float32)
        + bl_ref[...][None, :])


def _g_spec():
    return pl.BlockSpec((_RB, D), lambda i: (i, 0))


def _mm1(xp, W1, hist):
    return pl.pallas_call(
        _mm1_body,
        grid=(_GRID,),
        in_specs=[
            pl.BlockSpec((_RB, D), lambda i: (i, 0)),
            pl.BlockSpec((D, D), lambda i: (0, 0)),
            pl.BlockSpec((2, _RB), lambda i: (0, i)),
        ],
        out_specs=pl.BlockSpec((_RB, D), lambda i: (i, 0)),
        out_shape=jax.ShapeDtypeStruct((NP, D), jnp.float32),
    )(xp, W1, hist)


def _mid(g, y1, hist, b1, W2):
    return pl.pallas_call(
        _mid_body,
        grid=(_GRID,),
        in_specs=[
            _g_spec(),
            pl.BlockSpec((_RB, D), lambda i: (i, 0)),
            pl.BlockSpec((2, _RB), lambda i: (0, i)),
            pl.BlockSpec((D,), lambda i: (0,)),
            pl.BlockSpec((D, D), lambda i: (0, 0)),
        ],
        out_specs=[
            pl.BlockSpec((_RB, D), lambda i: (i, 0)),
            pl.BlockSpec((_RB, D), lambda i: (i, 0)),
        ],
        out_shape=[
            jax.ShapeDtypeStruct((NP, D), jnp.float32),
            jax.ShapeDtypeStruct((NP, D), jnp.float32),
        ],
    )(g, y1, hist, b1, W2)


def _fin(g, y2, hist, b2, x1, Wa, Wb, blin):
    return pl.pallas_call(
        _fin_body,
        grid=(_GRID,),
        in_specs=[
            _g_spec(),
            pl.BlockSpec((_RB, D), lambda i: (i, 0)),
            pl.BlockSpec((2, _RB), lambda i: (0, i)),
            pl.BlockSpec((D,), lambda i: (0,)),
            pl.BlockSpec((_RB, D), lambda i: (i, 0)),
            pl.BlockSpec((D, D), lambda i: (0, 0)),
            pl.BlockSpec((D, D), lambda i: (0, 0)),
            pl.BlockSpec((D,), lambda i: (0,)),
        ],
        out_specs=pl.BlockSpec((_RB, D), lambda i: (i, 0)),
        out_shape=jax.ShapeDtypeStruct((NP, D), jnp.float32),
    )(g, y2, hist, b2, x1, Wa, Wb, blin)


def kernel(x, edge_index, W1, b1, W2, b2, Wlin, blin):
    src = edge_index[0].astype(jnp.int32)
    dst = edge_index[1].astype(jnp.int32)
    e = src.shape[0]
    pad = NSUB * EPT - e
    fill = jnp.full((pad,), N, dtype=jnp.int32)
    srcp = jnp.concatenate([src, fill])
    dstp = jnp.concatenate([dst, fill])
    srcf = srcp.astype(jnp.float32).reshape(NSUB * NCHUNK, CHUNK)
    dstf = dstp.astype(jnp.float32).reshape(NSUB * NCHUNK, CHUNK)
    dst3 = dstp.reshape(NSUB, NCHUNK, CHUNK)
    xp = jnp.pad(x.astype(jnp.float32), ((0, NP - N), (0, 0)))
    hzeros = jnp.zeros((HSTRIPE,), jnp.float32)

    hist = _deg_kernel(dst3, hzeros)
    y1 = _mm1(xp, W1, hist)
    g1 = jnp.concatenate([a(y1, srcf, dstf) for a in _aggs], axis=0)
    x1, y2 = _mid(g1, y1, hist, b1, W2)
    g2 = jnp.concatenate([a(y2, srcf, dstf) for a in _aggs], axis=0)
    out = _fin(g2, y2, hist, b2, x1, Wlin[:D], Wlin[D:], blin)
    return out[:N]
